# Initial kernel scaffold; baseline (speedup 1.0000x reference)
import jax
import jax.numpy as jnp
from jax.experimental import pallas as pl


def kernel(x, t, Wr1, br1, Wr2, br2, W1_0, b1_0, W1_1, b1_1, W1_2, b1_2, W1_3, b1_3, W2_0, b2_0, W2_1, b2_1, W2_2, b2_2, W2_3, b2_3):
    raise NotImplementedError("write your pallas kernel here")


# speedup vs baseline: 1.5674x; 1.5674x over previous
import jax
import jax.numpy as jnp
from jax.experimental import pallas as pl

_N = 8192
_D = 768
_H = 64
_TILE = 1024


def _moe_kernel(x_ref, wr1_ref, br1_ref, wr2_ref, br2_ref,
                w10_ref, b10_ref, w11_ref, b11_ref, w12_ref, b12_ref,
                w13_ref, b13_ref,
                w20_ref, b20_ref, w21_ref, b21_ref, w22_ref, b22_ref,
                w23_ref, b23_ref, o_ref):
    x = x_ref[...]
    f32 = jnp.float32

    h = jnp.dot(x, wr1_ref[...], preferred_element_type=f32) + br1_ref[...]
    r = jnp.dot(h, wr2_ref[...], preferred_element_type=f32) + br2_ref[...]
    pick1 = r[:, 0:1] >= r[:, 1:2]

    sp = jax.nn.softplus

    y1 = sp(jnp.dot(x, w10_ref[...], preferred_element_type=f32) + b10_ref[...])
    y1 = sp(jnp.dot(y1, w11_ref[...], preferred_element_type=f32) + b11_ref[...])
    y1 = sp(jnp.dot(y1, w12_ref[...], preferred_element_type=f32) + b12_ref[...])
    y1 = jnp.dot(y1, w13_ref[...], preferred_element_type=f32) + b13_ref[...]

    y2 = sp(jnp.dot(x, w20_ref[...], preferred_element_type=f32) + b20_ref[...])
    y2 = sp(jnp.dot(y2, w21_ref[...], preferred_element_type=f32) + b21_ref[...])
    y2 = sp(jnp.dot(y2, w22_ref[...], preferred_element_type=f32) + b22_ref[...])
    y2 = jnp.dot(y2, w23_ref[...], preferred_element_type=f32) + b23_ref[...]

    o_ref[...] = jnp.where(pick1, y1, y2)


def kernel(x, t, Wr1, br1, Wr2, br2, W1_0, b1_0, W1_1, b1_1, W1_2, b1_2,
           W1_3, b1_3, W2_0, b2_0, W2_1, b2_1, W2_2, b2_2, W2_3, b2_3):
    del t

    wr1 = jnp.zeros((_D, 128), jnp.float32).at[:, :10].set(Wr1.T)
    br1p = jnp.zeros((1, 128), jnp.float32).at[0, :10].set(br1)
    wr2 = jnp.zeros((128, 8), jnp.float32).at[:10, :2].set(Wr2.T)
    br2p = jnp.zeros((1, 8), jnp.float32).at[0, :2].set(br2)

    rep2 = lambda i: (0, 0)
    tok = lambda i: (i, 0)

    def spec(shape, index_map):
        return pl.BlockSpec(shape, index_map)

    args = [
        (wr1, (_D, 128)), (br1p, (1, 128)),
        (wr2, (128, 8)), (br2p, (1, 8)),
        (W1_0.T, (_D, _H)), (b1_0[None, :], (1, _H)),
        (W1_1.T, (_H, _H)), (b1_1[None, :], (1, _H)),
        (W1_2.T, (_H, _H)), (b1_2[None, :], (1, _H)),
        (W1_3.T, (_H, _D)), (b1_3[None, :], (1, _D)),
        (W2_0.T, (_D, _H)), (b2_0[None, :], (1, _H)),
        (W2_1.T, (_H, _H)), (b2_1[None, :], (1, _H)),
        (W2_2.T, (_H, _H)), (b2_2[None, :], (1, _H)),
        (W2_3.T, (_H, _D)), (b2_3[None, :], (1, _D)),
    ]

    out = pl.pallas_call(
        _moe_kernel,
        grid=(_N // _TILE,),
        in_specs=[spec((_TILE, _D), tok)] + [spec(s, rep2) for _, s in args],
        out_specs=spec((_TILE, _D), tok),
        out_shape=jax.ShapeDtypeStruct((_N, _D), jnp.float32),
    )(x, *[a for a, _ in args])
    return out


# speedup vs baseline: 1.8560x; 1.1842x over previous
import jax
import jax.numpy as jnp
from jax.experimental import pallas as pl

_N = 8192
_D = 768
_H = 64
_TILE = 1024


def _moe_kernel(x_ref, wr1_ref, br1_ref, wr2_ref, br2_ref,
                w0_ref, b0_ref, w1_ref, b1_ref, w2_ref, b2_ref,
                w3_ref, b31_ref, b32_ref, o_ref):
    x = x_ref[...]
    f32 = jnp.float32

    hr = jnp.dot(x, wr1_ref[...], preferred_element_type=f32) + br1_ref[...]
    r = jnp.dot(hr, wr2_ref[...], preferred_element_type=f32) + br2_ref[...]
    pick1 = r[:, 0:1] >= r[:, 1:2]

    sp = jax.nn.softplus

    h = sp(jnp.dot(x, w0_ref[...], preferred_element_type=f32) + b0_ref[...])
    h = sp(jnp.dot(h, w1_ref[...], preferred_element_type=f32) + b1_ref[...])
    h = sp(jnp.dot(h, w2_ref[...], preferred_element_type=f32) + b2_ref[...])

    pick_f = jnp.where(pick1, 1.0, 0.0)
    cols = jax.lax.broadcasted_iota(jnp.int32, h.shape, 1)
    m = jnp.where(cols < _H, pick_f, 1.0 - pick_f)
    h = h * m
    y = jnp.dot(h, w3_ref[...], preferred_element_type=f32)
    o_ref[...] = y + jnp.where(pick1, b31_ref[...], b32_ref[...])


def kernel(x, t, Wr1, br1, Wr2, br2, W1_0, b1_0, W1_1, b1_1, W1_2, b1_2,
           W1_3, b1_3, W2_0, b2_0, W2_1, b2_1, W2_2, b2_2, W2_3, b2_3):
    del t

    wr1 = jnp.zeros((_D, 128), jnp.float32).at[:, :10].set(Wr1.T)
    br1p = jnp.zeros((1, 128), jnp.float32).at[0, :10].set(br1)
    wr2 = jnp.zeros((128, 8), jnp.float32).at[:10, :2].set(Wr2.T)
    br2p = jnp.zeros((1, 8), jnp.float32).at[0, :2].set(br2)

    h2 = 2 * _H
    w0 = jnp.concatenate([W1_0.T, W2_0.T], axis=1)
    b0 = jnp.concatenate([b1_0, b2_0])[None, :]
    w1 = jnp.zeros((h2, h2), jnp.float32)
    w1 = w1.at[:_H, :_H].set(W1_1.T).at[_H:, _H:].set(W2_1.T)
    b1 = jnp.concatenate([b1_1, b2_1])[None, :]
    w2 = jnp.zeros((h2, h2), jnp.float32)
    w2 = w2.at[:_H, :_H].set(W1_2.T).at[_H:, _H:].set(W2_2.T)
    b2 = jnp.concatenate([b1_2, b2_2])[None, :]
    w3 = jnp.concatenate([W1_3.T, W2_3.T], axis=0)
    b31 = b1_3[None, :]
    b32 = b2_3[None, :]

    rep2 = lambda i: (0, 0)
    tok = lambda i: (i, 0)

    args = [
        (wr1, (_D, 128)), (br1p, (1, 128)),
        (wr2, (128, 8)), (br2p, (1, 8)),
        (w0, (_D, h2)), (b0, (1, h2)),
        (w1, (h2, h2)), (b1, (1, h2)),
        (w2, (h2, h2)), (b2, (1, h2)),
        (w3, (h2, _D)), (b31, (1, _D)), (b32, (1, _D)),
    ]

    out = pl.pallas_call(
        _moe_kernel,
        grid=(_N // _TILE,),
        in_specs=[pl.BlockSpec((_TILE, _D), tok)]
        + [pl.BlockSpec(s, rep2) for _, s in args],
        out_specs=pl.BlockSpec((_TILE, _D), tok),
        out_shape=jax.ShapeDtypeStruct((_N, _D), jnp.float32),
    )(x, *[a for a, _ in args])
    return out


# speedup vs baseline: 1.8786x; 1.0121x over previous
import jax
import jax.numpy as jnp
from jax.experimental import pallas as pl

_N = 8192
_D = 768
_H = 64
_TILE = 1024


def _moe_kernel(x_ref, wr1_ref, br1_ref, wr2_ref, br2_ref,
                w0_ref, b0_ref, w1_ref, b1_ref, w2_ref, b2_ref,
                w3_ref, b31_ref, b32_ref, o_ref):
    x = x_ref[...]
    f32 = jnp.float32

    hr = jnp.dot(x, wr1_ref[...], preferred_element_type=f32) + br1_ref[...]
    r = jnp.dot(hr, wr2_ref[...], preferred_element_type=f32) + br2_ref[...]
    pick1 = r[:, 0:1] >= r[:, 1:2]

    sp = jax.nn.softplus

    h = sp(jnp.dot(x, w0_ref[...], preferred_element_type=f32) + b0_ref[...])
    h = sp(jnp.dot(h, w1_ref[...], preferred_element_type=f32) + b1_ref[...])
    h = sp(jnp.dot(h, w2_ref[...], preferred_element_type=f32) + b2_ref[...])

    pick_f = jnp.where(pick1, 1.0, 0.0)
    cols = jax.lax.broadcasted_iota(jnp.int32, h.shape, 1)
    m = jnp.where(cols < _H, pick_f, 1.0 - pick_f)
    h = h * m
    y = jnp.dot(h, w3_ref[...], preferred_element_type=f32)
    o_ref[...] = y + jnp.where(pick1, b31_ref[...], b32_ref[...])


def kernel(x, t, Wr1, br1, Wr2, br2, W1_0, b1_0, W1_1, b1_1, W1_2, b1_2,
           W1_3, b1_3, W2_0, b2_0, W2_1, b2_1, W2_2, b2_2, W2_3, b2_3):
    del t

    wr1 = jnp.zeros((_D, 16), jnp.float32).at[:, :10].set(Wr1.T)
    br1p = jnp.zeros((1, 16), jnp.float32).at[0, :10].set(br1)
    wr2 = jnp.zeros((16, 8), jnp.float32).at[:10, :2].set(Wr2.T)
    br2p = jnp.zeros((1, 8), jnp.float32).at[0, :2].set(br2)

    h2 = 2 * _H
    w0 = jnp.concatenate([W1_0.T, W2_0.T], axis=1)
    b0 = jnp.concatenate([b1_0, b2_0])[None, :]
    w1 = jnp.zeros((h2, h2), jnp.float32)
    w1 = w1.at[:_H, :_H].set(W1_1.T).at[_H:, _H:].set(W2_1.T)
    b1 = jnp.concatenate([b1_1, b2_1])[None, :]
    w2 = jnp.zeros((h2, h2), jnp.float32)
    w2 = w2.at[:_H, :_H].set(W1_2.T).at[_H:, _H:].set(W2_2.T)
    b2 = jnp.concatenate([b1_2, b2_2])[None, :]
    w3 = jnp.concatenate([W1_3.T, W2_3.T], axis=0)
    b31 = b1_3[None, :]
    b32 = b2_3[None, :]

    rep2 = lambda i: (0, 0)
    tok = lambda i: (i, 0)

    args = [
        (wr1, (_D, 16)), (br1p, (1, 16)),
        (wr2, (16, 8)), (br2p, (1, 8)),
        (w0, (_D, h2)), (b0, (1, h2)),
        (w1, (h2, h2)), (b1, (1, h2)),
        (w2, (h2, h2)), (b2, (1, h2)),
        (w3, (h2, _D)), (b31, (1, _D)), (b32, (1, _D)),
    ]

    out = pl.pallas_call(
        _moe_kernel,
        grid=(_N // _TILE,),
        in_specs=[pl.BlockSpec((_TILE, _D), tok)]
        + [pl.BlockSpec(s, rep2) for _, s in args],
        out_specs=pl.BlockSpec((_TILE, _D), tok),
        out_shape=jax.ShapeDtypeStruct((_N, _D), jnp.float32),
    )(x, *[a for a, _ in args])
    return out


# speedup vs baseline: 1.8799x; 1.0007x over previous
import jax
import jax.numpy as jnp
from jax.experimental import pallas as pl
from jax.experimental.pallas import tpu as pltpu

_N = 8192
_D = 768
_H = 64
_TILE = 1024


def _moe_kernel(x_ref, wr1_ref, br1_ref, wr2_ref, br2_ref,
                w0_ref, b0_ref, w1_ref, b1_ref, w2_ref, b2_ref,
                w3_ref, b31_ref, b32_ref, o_ref):
    x = x_ref[...]
    f32 = jnp.float32

    hr = jnp.dot(x, wr1_ref[...], preferred_element_type=f32) + br1_ref[...]
    r = jnp.dot(hr, wr2_ref[...], preferred_element_type=f32) + br2_ref[...]
    pick1 = r[:, 0:1] >= r[:, 1:2]

    sp = jax.nn.softplus

    h = sp(jnp.dot(x, w0_ref[...], preferred_element_type=f32) + b0_ref[...])
    h = sp(jnp.dot(h, w1_ref[...], preferred_element_type=f32) + b1_ref[...])
    h = sp(jnp.dot(h, w2_ref[...], preferred_element_type=f32) + b2_ref[...])

    pick_f = jnp.where(pick1, 1.0, 0.0)
    cols = jax.lax.broadcasted_iota(jnp.int32, h.shape, 1)
    m = jnp.where(cols < _H, pick_f, 1.0 - pick_f)
    h = h * m
    y = jnp.dot(h, w3_ref[...], preferred_element_type=f32)
    o_ref[...] = y + jnp.where(pick1, b31_ref[...], b32_ref[...])


def kernel(x, t, Wr1, br1, Wr2, br2, W1_0, b1_0, W1_1, b1_1, W1_2, b1_2,
           W1_3, b1_3, W2_0, b2_0, W2_1, b2_1, W2_2, b2_2, W2_3, b2_3):
    del t

    wr1 = jnp.zeros((_D, 16), jnp.float32).at[:, :10].set(Wr1.T)
    br1p = jnp.zeros((1, 16), jnp.float32).at[0, :10].set(br1)
    wr2 = jnp.zeros((16, 8), jnp.float32).at[:10, :2].set(Wr2.T)
    br2p = jnp.zeros((1, 8), jnp.float32).at[0, :2].set(br2)

    h2 = 2 * _H
    w0 = jnp.concatenate([W1_0.T, W2_0.T], axis=1)
    b0 = jnp.concatenate([b1_0, b2_0])[None, :]
    w1 = jnp.zeros((h2, h2), jnp.float32)
    w1 = w1.at[:_H, :_H].set(W1_1.T).at[_H:, _H:].set(W2_1.T)
    b1 = jnp.concatenate([b1_1, b2_1])[None, :]
    w2 = jnp.zeros((h2, h2), jnp.float32)
    w2 = w2.at[:_H, :_H].set(W1_2.T).at[_H:, _H:].set(W2_2.T)
    b2 = jnp.concatenate([b1_2, b2_2])[None, :]
    w3 = jnp.concatenate([W1_3.T, W2_3.T], axis=0)
    b31 = b1_3[None, :]
    b32 = b2_3[None, :]

    rep2 = lambda i: (0, 0)
    tok = lambda i: (i, 0)

    args = [
        (wr1, (_D, 16)), (br1p, (1, 16)),
        (wr2, (16, 8)), (br2p, (1, 8)),
        (w0, (_D, h2)), (b0, (1, h2)),
        (w1, (h2, h2)), (b1, (1, h2)),
        (w2, (h2, h2)), (b2, (1, h2)),
        (w3, (h2, _D)), (b31, (1, _D)), (b32, (1, _D)),
    ]

    out = pl.pallas_call(
        _moe_kernel,
        grid=(_N // _TILE,),
        in_specs=[pl.BlockSpec((_TILE, _D), tok)]
        + [pl.BlockSpec(s, rep2) for _, s in args],
        out_specs=pl.BlockSpec((_TILE, _D), tok),
        out_shape=jax.ShapeDtypeStruct((_N, _D), jnp.float32),
        compiler_params=pltpu.CompilerParams(
            dimension_semantics=("parallel",)),
    )(x, *[a for a, _ in args])
    return out


# speedup vs baseline: 1.9066x; 1.0142x over previous
import jax
import jax.numpy as jnp
from jax.experimental import pallas as pl
from jax.experimental.pallas import tpu as pltpu

_N = 8192
_D = 768
_H = 64
_TILE = 1024


def _moe_kernel(x_ref, wr1_ref, br1_ref, wr2_ref, br2_ref,
                w0_ref, b0_ref, w1_ref, b1_ref, w2_ref, b2_ref,
                w3_ref, b31_ref, b32_ref, o_ref):
    x = x_ref[...]
    f32 = jnp.float32

    hr = jnp.dot(x, wr1_ref[...], preferred_element_type=f32) + br1_ref[...]
    r = jnp.dot(hr, wr2_ref[...], preferred_element_type=f32) + br2_ref[...]
    pick1 = r[:, 0:1] >= r[:, 1:2]

    sp = jax.nn.softplus
    bf16 = jnp.bfloat16

    xb = x.astype(bf16)
    h = sp(jnp.dot(xb, w0_ref[...], preferred_element_type=f32) + b0_ref[...])
    h = sp(jnp.dot(h.astype(bf16), w1_ref[...], preferred_element_type=f32)
           + b1_ref[...])
    h = sp(jnp.dot(h.astype(bf16), w2_ref[...], preferred_element_type=f32)
           + b2_ref[...])

    pick_f = jnp.where(pick1, 1.0, 0.0)
    cols = jax.lax.broadcasted_iota(jnp.int32, h.shape, 1)
    m = jnp.where(cols < _H, pick_f, 1.0 - pick_f)
    h = h * m
    y = jnp.dot(h.astype(bf16), w3_ref[...], preferred_element_type=f32)
    o_ref[...] = y + jnp.where(pick1, b31_ref[...], b32_ref[...])


def kernel(x, t, Wr1, br1, Wr2, br2, W1_0, b1_0, W1_1, b1_1, W1_2, b1_2,
           W1_3, b1_3, W2_0, b2_0, W2_1, b2_1, W2_2, b2_2, W2_3, b2_3):
    del t

    wr1 = jnp.zeros((_D, 16), jnp.float32).at[:, :10].set(Wr1.T)
    br1p = jnp.zeros((1, 16), jnp.float32).at[0, :10].set(br1)
    wr2 = jnp.zeros((16, 8), jnp.float32).at[:10, :2].set(Wr2.T)
    br2p = jnp.zeros((1, 8), jnp.float32).at[0, :2].set(br2)

    h2 = 2 * _H
    bf = jnp.bfloat16
    w0 = jnp.concatenate([W1_0.T, W2_0.T], axis=1).astype(bf)
    b0 = jnp.concatenate([b1_0, b2_0])[None, :]
    w1 = jnp.zeros((h2, h2), jnp.float32)
    w1 = w1.at[:_H, :_H].set(W1_1.T).at[_H:, _H:].set(W2_1.T).astype(bf)
    b1 = jnp.concatenate([b1_1, b2_1])[None, :]
    w2 = jnp.zeros((h2, h2), jnp.float32)
    w2 = w2.at[:_H, :_H].set(W1_2.T).at[_H:, _H:].set(W2_2.T).astype(bf)
    b2 = jnp.concatenate([b1_2, b2_2])[None, :]
    w3 = jnp.concatenate([W1_3.T, W2_3.T], axis=0).astype(bf)
    b31 = b1_3[None, :]
    b32 = b2_3[None, :]

    rep2 = lambda i: (0, 0)
    tok = lambda i: (i, 0)

    args = [
        (wr1, (_D, 16)), (br1p, (1, 16)),
        (wr2, (16, 8)), (br2p, (1, 8)),
        (w0, (_D, h2)), (b0, (1, h2)),
        (w1, (h2, h2)), (b1, (1, h2)),
        (w2, (h2, h2)), (b2, (1, h2)),
        (w3, (h2, _D)), (b31, (1, _D)), (b32, (1, _D)),
    ]

    out = pl.pallas_call(
        _moe_kernel,
        grid=(_N // _TILE,),
        in_specs=[pl.BlockSpec((_TILE, _D), tok)]
        + [pl.BlockSpec(s, rep2) for _, s in args],
        out_specs=pl.BlockSpec((_TILE, _D), tok),
        out_shape=jax.ShapeDtypeStruct((_N, _D), jnp.float32),
        compiler_params=pltpu.CompilerParams(
            dimension_semantics=("parallel",)),
    )(x, *[a for a, _ in args])
    return out


# speedup vs baseline: 1.9849x; 1.0411x over previous
import jax
import jax.numpy as jnp
from jax.experimental import pallas as pl
from jax.experimental.pallas import tpu as pltpu

_N = 8192
_D = 768
_H = 64
_TILE = 1024

_TRANS_RHS = (((1,), (1,)), ((), ()))


def _dotn(a, b):
    return jax.lax.dot_general(a, b, _TRANS_RHS,
                               preferred_element_type=jnp.float32)


def _moe_kernel(x_ref, wr1_ref, br1_ref, wr2_ref, br2_ref,
                w0_ref, b0_ref, w1_ref, b1_ref, w2_ref, b2_ref,
                w3_ref, b31_ref, b32_ref, o_ref):
    x = x_ref[...]

    hr = _dotn(x, wr1_ref[...]) + br1_ref[...]
    r = _dotn(hr, wr2_ref[...]) + br2_ref[...]
    pick1 = r[:, 0:1] >= r[:, 1:2]

    sp = jax.nn.softplus
    bf16 = jnp.bfloat16

    h = sp(_dotn(x.astype(bf16), w0_ref[...]) + b0_ref[...])
    h = sp(_dotn(h.astype(bf16), w1_ref[...]) + b1_ref[...])
    h = sp(_dotn(h.astype(bf16), w2_ref[...]) + b2_ref[...])

    pick_f = jnp.where(pick1, 1.0, 0.0)
    cols = jax.lax.broadcasted_iota(jnp.int32, h.shape, 1)
    m = jnp.where(cols < _H, pick_f, 1.0 - pick_f)
    h = h * m
    y = _dotn(h.astype(bf16), w3_ref[...])
    o_ref[...] = y + jnp.where(pick1, b31_ref[...], b32_ref[...])


def kernel(x, t, Wr1, br1, Wr2, br2, W1_0, b1_0, W1_1, b1_1, W1_2, b1_2,
           W1_3, b1_3, W2_0, b2_0, W2_1, b2_1, W2_2, b2_2, W2_3, b2_3):
    del t

    f32 = jnp.float32
    bf = jnp.bfloat16
    h2 = 2 * _H

    wr1 = jnp.zeros((16, _D), f32).at[:10, :].set(Wr1)
    br1p = jnp.zeros((1, 16), f32).at[0, :10].set(br1)
    wr2 = jnp.zeros((8, 16), f32).at[:2, :10].set(Wr2)
    br2p = jnp.zeros((1, 8), f32).at[0, :2].set(br2)

    w0 = jnp.concatenate([W1_0, W2_0], axis=0).astype(bf)
    b0 = jnp.concatenate([b1_0, b2_0])[None, :]
    zh = jnp.zeros((_H, _H), f32)
    w1 = jnp.concatenate(
        [jnp.concatenate([W1_1, zh], axis=1),
         jnp.concatenate([zh, W2_1], axis=1)], axis=0).astype(bf)
    b1 = jnp.concatenate([b1_1, b2_1])[None, :]
    w2 = jnp.concatenate(
        [jnp.concatenate([W1_2, zh], axis=1),
         jnp.concatenate([zh, W2_2], axis=1)], axis=0).astype(bf)
    b2 = jnp.concatenate([b1_2, b2_2])[None, :]
    w3 = jnp.concatenate([W1_3, W2_3], axis=1).astype(bf)
    b31 = b1_3[None, :]
    b32 = b2_3[None, :]

    rep2 = lambda i: (0, 0)
    tok = lambda i: (i, 0)

    args = [
        (wr1, (16, _D)), (br1p, (1, 16)),
        (wr2, (8, 16)), (br2p, (1, 8)),
        (w0, (h2, _D)), (b0, (1, h2)),
        (w1, (h2, h2)), (b1, (1, h2)),
        (w2, (h2, h2)), (b2, (1, h2)),
        (w3, (_D, h2)), (b31, (1, _D)), (b32, (1, _D)),
    ]

    out = pl.pallas_call(
        _moe_kernel,
        grid=(_N // _TILE,),
        in_specs=[pl.BlockSpec((_TILE, _D), tok)]
        + [pl.BlockSpec(s, rep2) for _, s in args],
        out_specs=pl.BlockSpec((_TILE, _D), tok),
        out_shape=jax.ShapeDtypeStruct((_N, _D), jnp.float32),
        compiler_params=pltpu.CompilerParams(
            dimension_semantics=("parallel",)),
    )(x, *[a for a, _ in args])
    return out


# speedup vs baseline: 2.8089x; 1.4151x over previous
import jax
import jax.numpy as jnp
from jax.experimental import pallas as pl
from jax.experimental.pallas import tpu as pltpu

_N = 8192
_D = 768
_H = 64
_TILE = 1024

_TRANS_RHS = (((1,), (1,)), ((), ()))


def _dotn(a, b):
    return jax.lax.dot_general(a, b, _TRANS_RHS,
                               preferred_element_type=jnp.float32)


def _moe_kernel(x_ref, wr1_ref, br1_ref, wr2_ref, br2_ref,
                w10_ref, b10_ref, w11_ref, b11_ref, w12_ref, b12_ref,
                w13_ref, b13_ref,
                w20_ref, b20_ref, w21_ref, b21_ref, w22_ref, b22_ref,
                w23_ref, b23_ref, o_ref,
                wr1s, br1s, wr2s, br2s, w0s, b0s, w1s, b1s, w2s, b2s, w3s):
    f32 = jnp.float32
    bf16 = jnp.bfloat16

    @pl.when(pl.program_id(0) == 0)
    def _pack():
        cat = jnp.concatenate
        wr1s[...] = cat([wr1_ref[...], jnp.zeros((6, _D), f32)], axis=0)
        br1s[...] = cat([br1_ref[...], jnp.zeros((1, 6), f32)], axis=1)
        wr2s[...] = cat(
            [cat([wr2_ref[...], jnp.zeros((2, 6), f32)], axis=1),
             jnp.zeros((6, 16), f32)], axis=0)
        br2s[...] = cat([br2_ref[...], jnp.zeros((1, 6), f32)], axis=1)
        w0s[...] = cat([w10_ref[...], w20_ref[...]], axis=0).astype(bf16)
        zh = jnp.zeros((_H, _H), f32)
        w1s[...] = cat(
            [cat([w11_ref[...], zh], axis=1),
             cat([zh, w21_ref[...]], axis=1)], axis=0).astype(bf16)
        w2s[...] = cat(
            [cat([w12_ref[...], zh], axis=1),
             cat([zh, w22_ref[...]], axis=1)], axis=0).astype(bf16)
        w3s[...] = cat([w13_ref[...], w23_ref[...]], axis=1).astype(bf16)
        b0s[...] = cat([b10_ref[...], b20_ref[...]], axis=1)
        b1s[...] = cat([b11_ref[...], b21_ref[...]], axis=1)
        b2s[...] = cat([b12_ref[...], b22_ref[...]], axis=1)

    x = x_ref[...]

    hr = _dotn(x, wr1s[...]) + br1s[...]
    r = _dotn(hr, wr2s[...]) + br2s[...]
    pick1 = r[:, 0:1] >= r[:, 1:2]

    sp = jax.nn.softplus

    h = sp(_dotn(x.astype(bf16), w0s[...]) + b0s[...])
    h = sp(_dotn(h.astype(bf16), w1s[...]) + b1s[...])
    h = sp(_dotn(h.astype(bf16), w2s[...]) + b2s[...])

    pick_f = jnp.where(pick1, 1.0, 0.0)
    cols = jax.lax.broadcasted_iota(jnp.int32, h.shape, 1)
    m = jnp.where(cols < _H, pick_f, 1.0 - pick_f)
    h = h * m
    y = _dotn(h.astype(bf16), w3s[...])
    o_ref[...] = y + jnp.where(pick1, b13_ref[...], b23_ref[...])


def kernel(x, t, Wr1, br1, Wr2, br2, W1_0, b1_0, W1_1, b1_1, W1_2, b1_2,
           W1_3, b1_3, W2_0, b2_0, W2_1, b2_1, W2_2, b2_2, W2_3, b2_3):
    del t

    f32 = jnp.float32
    bf16 = jnp.bfloat16
    h2 = 2 * _H

    rep2 = lambda i: (0, 0)
    tok = lambda i: (i, 0)

    args = [
        (Wr1, (10, _D)), (br1[None, :], (1, 10)),
        (Wr2, (2, 10)), (br2[None, :], (1, 2)),
        (W1_0, (_H, _D)), (b1_0[None, :], (1, _H)),
        (W1_1, (_H, _H)), (b1_1[None, :], (1, _H)),
        (W1_2, (_H, _H)), (b1_2[None, :], (1, _H)),
        (W1_3, (_D, _H)), (b1_3[None, :], (1, _D)),
        (W2_0, (_H, _D)), (b2_0[None, :], (1, _H)),
        (W2_1, (_H, _H)), (b2_1[None, :], (1, _H)),
        (W2_2, (_H, _H)), (b2_2[None, :], (1, _H)),
        (W2_3, (_D, _H)), (b2_3[None, :], (1, _D)),
    ]

    out = pl.pallas_call(
        _moe_kernel,
        grid=(_N // _TILE,),
        in_specs=[pl.BlockSpec((_TILE, _D), tok)]
        + [pl.BlockSpec(s, rep2) for _, s in args],
        out_specs=pl.BlockSpec((_TILE, _D), tok),
        out_shape=jax.ShapeDtypeStruct((_N, _D), jnp.float32),
        scratch_shapes=[
            pltpu.VMEM((16, _D), f32),
            pltpu.VMEM((1, 16), f32),
            pltpu.VMEM((8, 16), f32),
            pltpu.VMEM((1, 8), f32),
            pltpu.VMEM((h2, _D), bf16),
            pltpu.VMEM((1, h2), f32),
            pltpu.VMEM((h2, h2), bf16),
            pltpu.VMEM((1, h2), f32),
            pltpu.VMEM((h2, h2), bf16),
            pltpu.VMEM((1, h2), f32),
            pltpu.VMEM((_D, h2), bf16),
        ],
    )(x, *[a for a, _ in args])
    return out
